# baseline (speedup 1.0000x reference)
import jax
import jax.numpy as jnp
from jax.experimental import pallas as pl
from jax.experimental.pallas import tpu as pltpu

TOPK = 16
N = 4096
D = 512
H = 128
BLOCK = 128


def _proj_kernel(x_ref, w_ref, b_ref, h_ref):
    h_ref[...] = (
        jnp.dot(x_ref[...], w_ref[...], preferred_element_type=jnp.float32)
        + b_ref[...]
    )


def _adj_topk_kernel(hblk_ref, hall_ref, adj_ref, idx_ref):
    hb = hblk_ref[...]
    ha = hall_ref[...]
    sim = jax.lax.dot_general(
        hb, ha, (((1,), (1,)), ((), ())), preferred_element_type=jnp.float32
    )
    m = jnp.max(sim, axis=1, keepdims=True)
    e = jnp.exp(sim - m)
    s = jnp.sum(e, axis=1, keepdims=True)
    adj = e / s
    adj_ref[...] = adj

    iota = jax.lax.broadcasted_iota(jnp.int32, (BLOCK, N), 1)
    vals = adj
    cols = []
    for _ in range(TOPK):
        mj = jnp.max(vals, axis=1, keepdims=True)
        idx = jnp.min(jnp.where(vals == mj, iota, N), axis=1)
        cols.append(idx)
        vals = jnp.where(iota == idx[:, None], -1.0, vals)
    idx_ref[...] = jnp.stack(cols, axis=1)


def kernel(x, W, b):
    h = pl.pallas_call(
        _proj_kernel,
        out_shape=jax.ShapeDtypeStruct((N, H), jnp.float32),
    )(x, W, b.reshape(1, H))

    grid = (N // BLOCK,)
    adj, idx = pl.pallas_call(
        _adj_topk_kernel,
        grid=grid,
        in_specs=[
            pl.BlockSpec((BLOCK, H), lambda i: (i, 0)),
            pl.BlockSpec((N, H), lambda i: (0, 0)),
        ],
        out_specs=[
            pl.BlockSpec((BLOCK, N), lambda i: (i, 0)),
            pl.BlockSpec((BLOCK, TOPK), lambda i: (i, 0)),
        ],
        out_shape=[
            jax.ShapeDtypeStruct((N, N), jnp.float32),
            jax.ShapeDtypeStruct((N, TOPK), jnp.int32),
        ],
    )(h, h)

    src = jnp.repeat(jnp.arange(N, dtype=jnp.int64), TOPK)
    dst = idx.reshape(-1).astype(jnp.int64)
    edge_index = jnp.stack([src, dst], axis=0)
    return adj, edge_index


# speedup vs baseline: 1.2385x; 1.2385x over previous
import jax
import jax.numpy as jnp
from jax.experimental import pallas as pl
from jax.experimental.pallas import tpu as pltpu

TOPK = 16
N = 4096
D = 512
H = 128
BLOCK = 256


def _proj_kernel(x_ref, w_ref, b_ref, h_ref):
    h_ref[...] = (
        jnp.dot(x_ref[...], w_ref[...], preferred_element_type=jnp.float32)
        + b_ref[...]
    )


def _adj_topk_kernel(hblk_ref, hall_ref, adj_ref, idx_ref):
    hb = hblk_ref[...]
    ha = hall_ref[...]
    sim = jax.lax.dot_general(
        hb, ha, (((1,), (1,)), ((), ())), preferred_element_type=jnp.float32
    )
    m = jnp.max(sim, axis=1, keepdims=True)
    e = jnp.exp(sim - m)
    s = jnp.sum(e, axis=1, keepdims=True)
    adj = e / s
    adj_ref[...] = adj

    iota = jax.lax.broadcasted_iota(jnp.int32, (BLOCK, N), 1)
    vals = adj
    cols = []
    for _ in range(TOPK):
        idx = jnp.argmax(vals, axis=1).astype(jnp.int32)
        cols.append(idx)
        vals = jnp.where(iota == idx[:, None], -1.0, vals)
    idx_ref[...] = jnp.stack(cols, axis=1)


def kernel(x, W, b):
    h = pl.pallas_call(
        _proj_kernel,
        out_shape=jax.ShapeDtypeStruct((N, H), jnp.float32),
    )(x, W, b.reshape(1, H))

    grid = (N // BLOCK,)
    adj, idx = pl.pallas_call(
        _adj_topk_kernel,
        grid=grid,
        in_specs=[
            pl.BlockSpec((BLOCK, H), lambda i: (i, 0)),
            pl.BlockSpec((N, H), lambda i: (0, 0)),
        ],
        out_specs=[
            pl.BlockSpec((BLOCK, N), lambda i: (i, 0)),
            pl.BlockSpec((BLOCK, TOPK), lambda i: (i, 0)),
        ],
        out_shape=[
            jax.ShapeDtypeStruct((N, N), jnp.float32),
            jax.ShapeDtypeStruct((N, TOPK), jnp.int32),
        ],
    )(h, h)

    src = jnp.repeat(jnp.arange(N, dtype=jnp.int64), TOPK)
    dst = idx.reshape(-1).astype(jnp.int64)
    edge_index = jnp.stack([src, dst], axis=0)
    return adj, edge_index
